# baseline (speedup 1.0000x reference)
import functools

import jax
import jax.numpy as jnp
from jax import lax
from jax.experimental import pallas as pl
from jax.experimental.pallas import tpu as pltpu
from jax.experimental.pallas import tpu_sc as plsc

_NC = 2
_NS = 16
_NW = _NC * _NS


@functools.cache
def _build_transpose(V, D):
    L = 16
    CH = 512
    n_ch = V // CH
    tail = V - n_ch * CH
    mesh = plsc.VectorSubcoreMesh(core_axis_name="c", subcore_axis_name="s")

    @functools.partial(
        pl.kernel,
        out_type=jax.ShapeDtypeStruct((V * D // 128, 128), jnp.float32),
        mesh=mesh,
        scratch_types=[
            pltpu.VMEM((2, D, CH), jnp.float32),
            pltpu.VMEM((2, CH * D // 128, 128), jnp.float32),
            pltpu.VMEM((max(8, (V - (V // CH) * CH) * D // 128), 128),
                       jnp.float32),
            pltpu.SemaphoreType.DMA,
            pltpu.SemaphoreType.DMA,
            pltpu.SemaphoreType.DMA,
            pltpu.SemaphoreType.DMA,
        ],
        compiler_params=pltpu.CompilerParams(needs_layout_passes=False),
    )
    def tr_kernel(tt_hbm, tail_hbm, out_hbm, in_v, out_v, tail_v,
                  sem_i0, sem_i1, sem_o0, sem_o1):
        wid = lax.axis_index("s") * _NC + lax.axis_index("c")
        sems_i = (sem_i0, sem_i1)
        sems_o = (sem_o0, sem_o1)
        orows = CH * D // 128

        def in_desc(ch, p, sem):
            return pltpu.make_async_copy(
                tt_hbm.at[:, pl.ds(pl.multiple_of(ch * CH, 128), CH)],
                in_v.at[p], sem)

        def out_desc(ch, p, sem):
            return pltpu.make_async_copy(
                out_v.at[p],
                out_hbm.at[pl.ds(pl.multiple_of(ch * orows, 8), orows)], sem)

        def per_parity(c, fn):
            @pl.when(c % 2 == 0)
            def _():
                fn(0)

            @pl.when(c % 2 == 1)
            def _():
                fn(1)

        def compute(r, p):
            iot = lax.iota(jnp.int32, L)
            cidx0 = iot % D if D < L else iot

            def qgrp(qm, carry):
                for m in range(128 // L):
                    lam0 = m * L
                    cvec = (lam0 + iot) % D
                    avec = (lam0 + iot) // D
                    jvec = qm * (128 // D) + avec
                    seg = plsc.load_gather(in_v.at[p], [cvec, jvec])
                    out_v[p, qm, pl.ds(lam0, L)] = seg
                return carry

            lax.fori_loop(0, orows, qgrp, 0)

        n_r = (n_ch - wid + _NW - 1) // _NW

        @pl.when(n_r > 0)
        def _():
            per_parity(0, lambda p: in_desc(wid, p, sems_i[p]).start())

            def r_body(r, carry):
                ch = wid + r * _NW

                @pl.when(r + 1 < n_r)
                def _():
                    per_parity(r + 1, lambda p: in_desc(
                        ch + _NW, p, sems_i[p]).start())

                per_parity(r, lambda p: in_desc(ch, p, sems_i[p]).wait())

                @pl.when(r >= 2)
                def _():
                    per_parity(r - 2, lambda p: out_desc(
                        ch - 2 * _NW, p, sems_o[p]).wait())

                per_parity(r, lambda p: compute(r, p))
                per_parity(r, lambda p: out_desc(ch, p, sems_o[p]).start())
                return carry

            lax.fori_loop(0, n_r, r_body, 0)
            per_parity(n_r - 2, lambda p: out_desc(
                wid + (n_r - 2) * _NW, p, sems_o[p]).wait())
            per_parity(n_r - 1, lambda p: out_desc(
                wid + (n_r - 1) * _NW, p, sems_o[p]).wait())

        if tail:
            @pl.when(wid == 0)
            def _():
                trows = tail * D // 128
                pltpu.sync_copy(tail_hbm, tail_v.at[pl.ds(0, trows)])
                pltpu.sync_copy(
                    tail_v.at[pl.ds(0, trows)],
                    out_hbm.at[pl.ds(n_ch * CH * D // 128, trows)])

    return tr_kernel


@functools.cache
def _build(B, NNZ, V, D):
    L = 16
    bw = B // _NW
    rpq = 128 // D
    shf = (rpq - 1).bit_length()
    dshf = (D - 1).bit_length()
    mesh = plsc.VectorSubcoreMesh(core_axis_name="c", subcore_axis_name="s")

    @functools.partial(
        pl.kernel,
        out_type=jax.ShapeDtypeStruct((NNZ, D, B), jnp.float32),
        mesh=mesh,
        scratch_types=[
            pltpu.VMEM((NNZ, bw), jnp.int32),
            pltpu.VMEM((NNZ, bw), jnp.float32),
            pltpu.VMEM((NNZ * bw,), jnp.int32),
            pltpu.VMEM((NNZ, bw), jnp.int32),
            pltpu.VMEM((2, bw, 128), jnp.float32),
            pltpu.VMEM((2, D, bw), jnp.float32),
            pltpu.SemaphoreType.DMA,
            pltpu.SemaphoreType.DMA,
            pltpu.SemaphoreType.DMA,
            pltpu.SemaphoreType.DMA,
        ],
        compiler_params=pltpu.CompilerParams(needs_layout_passes=False),
    )
    def sc_kernel(xt_hbm, vt_hbm, table_hbm, out_hbm,
                  x_v, val_v, q_v, off_v, quad_v, out_v,
                  sem_g0, sem_g1, sem_o0, sem_o1):
        wid = lax.axis_index("s") * _NC + lax.axis_index("c")
        b0 = pl.multiple_of(wid * bw, 128)
        pltpu.sync_copy(xt_hbm.at[:, pl.ds(b0, bw)], x_v)
        pltpu.sync_copy(vt_hbm.at[:, pl.ds(b0, bw)], val_v)

        def fmt_body(n, carry):
            for k in range(bw // L):
                x16 = x_v[n, pl.ds(k * L, L)]
                q_v[pl.ds(n * bw + k * L, L)] = lax.shift_right_logical(x16, shf)
                off_v[n, pl.ds(k * L, L)] = lax.shift_left(jnp.bitwise_and(x16, rpq - 1), dshf)
            return carry

        lax.fori_loop(0, NNZ, fmt_body, 0)

        def gather_desc(n, p, sem):
            return pltpu.make_async_copy(
                table_hbm.at[q_v.at[pl.ds(n * bw, bw)]],
                quad_v.at[p],
                sem,
            )

        def out_desc(n, p, sem):
            return pltpu.make_async_copy(
                out_v.at[p],
                out_hbm.at[n, :, pl.ds(b0, bw)],
                sem,
            )

        def per_parity(c, fn):
            @pl.when(c % 2 == 0)
            def _():
                fn(0)

            @pl.when(c % 2 == 1)
            def _():
                fn(1)

        sems_g = (sem_g0, sem_g1)
        sems_o = (sem_o0, sem_o1)

        def compute_p(n, p):
            row0 = lax.iota(jnp.int32, L)
            for k in range(bw // L):
                off16 = off_v[n, pl.ds(k * L, L)]
                val16 = val_v[n, pl.ds(k * L, L)]
                rows16 = row0 + k * L
                for c in range(D):
                    seg = plsc.load_gather(
                        quad_v.at[p], [rows16, off16 + c])
                    out_v[p, c, pl.ds(k * L, L)] = seg * val16

        def compute(n):
            per_parity(n, lambda p: compute_p(n, p))

        per_parity(0, lambda p: gather_desc(0, p, sems_g[p]).start())

        def n_body(n, carry):
            @pl.when(n < NNZ - 1)
            def _():
                per_parity(n + 1,
                           lambda p: gather_desc(n + 1, p, sems_g[p]).start())

            per_parity(n, lambda p: gather_desc(n, p, sems_g[p]).wait())

            @pl.when(n >= 2)
            def _():
                per_parity(n - 2,
                           lambda p: out_desc(n - 2, p, sems_o[p]).wait())

            compute(n)
            per_parity(n, lambda p: out_desc(n, p, sems_o[p]).start())
            return carry

        lax.fori_loop(0, NNZ, n_body, 0)
        per_parity(NNZ - 2, lambda p: out_desc(NNZ - 2, p, sems_o[p]).wait())
        per_parity(NNZ - 1, lambda p: out_desc(NNZ - 1, p, sems_o[p]).wait())

    return sc_kernel


def kernel(x, x_val, table):
    B, NNZ = x.shape
    V, D = table.shape
    rpq = 128 // D
    xt = jnp.transpose(x).astype(jnp.int32)
    vt = jnp.transpose(x_val)
    tt = jnp.transpose(table)
    n_ch = V // 512
    tail = V - n_ch * 512
    if tail:
        tail_in = table[n_ch * 512:].reshape(tail * D // 128, 128)
    else:
        tail_in = jnp.zeros((8, 128), jnp.float32)
    t4 = _build_transpose(V, D)(tt, tail_in)
    out_t = _build(B, NNZ, V, D)(xt, vt, t4)
    return jnp.transpose(out_t, (2, 0, 1))


# speedup vs baseline: 1.4366x; 1.4366x over previous
import functools

import jax
import jax.numpy as jnp
from jax import lax
from jax.experimental import pallas as pl
from jax.experimental.pallas import tpu as pltpu
from jax.experimental.pallas import tpu_sc as plsc

_NC = 2
_NS = 16
_NW = _NC * _NS


@functools.cache
def _build_transpose(V, D):
    L = 16
    CH = 512
    n_ch = V // CH
    tail = V - n_ch * CH
    mesh = plsc.VectorSubcoreMesh(core_axis_name="c", subcore_axis_name="s")

    @functools.partial(
        pl.kernel,
        out_type=jax.ShapeDtypeStruct((V * D // 128, 128), jnp.float32),
        mesh=mesh,
        scratch_types=[
            pltpu.VMEM((2, D, CH), jnp.float32),
            pltpu.VMEM((2, CH * D // 128, 128), jnp.float32),
            pltpu.VMEM((max(8, (V - (V // CH) * CH) * D // 128), 128),
                       jnp.float32),
            pltpu.SemaphoreType.DMA,
            pltpu.SemaphoreType.DMA,
            pltpu.SemaphoreType.DMA,
            pltpu.SemaphoreType.DMA,
        ],
        compiler_params=pltpu.CompilerParams(needs_layout_passes=False),
    )
    def tr_kernel(tt_hbm, tail_hbm, out_hbm, in_v, out_v, tail_v,
                  sem_i0, sem_i1, sem_o0, sem_o1):
        wid = lax.axis_index("s") * _NC + lax.axis_index("c")
        sems_i = (sem_i0, sem_i1)
        sems_o = (sem_o0, sem_o1)
        orows = CH * D // 128

        def in_desc(ch, p, sem):
            return pltpu.make_async_copy(
                tt_hbm.at[:, pl.ds(pl.multiple_of(ch * CH, 128), CH)],
                in_v.at[p], sem)

        def out_desc(ch, p, sem):
            return pltpu.make_async_copy(
                out_v.at[p],
                out_hbm.at[pl.ds(pl.multiple_of(ch * orows, 8), orows)], sem)

        def per_parity(c, fn):
            @pl.when(c % 2 == 0)
            def _():
                fn(0)

            @pl.when(c % 2 == 1)
            def _():
                fn(1)

        iot = lax.iota(jnp.int32, L)
        cvecs = [(m * L + iot) % D for m in range(128 // L)]
        avecs = [(m * L + iot) // D for m in range(128 // L)]
        UQ = 4

        def compute(r, p):
            @plsc.parallel_loop(0, orows, step=UQ)
            def qgrp(qm0):
                for u in range(UQ):
                    qm = qm0 + u
                    jbase = qm * (128 // D)
                    for m in range(128 // L):
                        seg = plsc.load_gather(
                            in_v.at[p], [cvecs[m], avecs[m] + jbase])
                        out_v[p, qm, pl.ds(m * L, L)] = seg

        n_r = (n_ch - wid + _NW - 1) // _NW

        @pl.when(n_r > 0)
        def _():
            per_parity(0, lambda p: in_desc(wid, p, sems_i[p]).start())

            def r_body(r, carry):
                ch = wid + r * _NW

                @pl.when(r + 1 < n_r)
                def _():
                    per_parity(r + 1, lambda p: in_desc(
                        ch + _NW, p, sems_i[p]).start())

                per_parity(r, lambda p: in_desc(ch, p, sems_i[p]).wait())

                @pl.when(r >= 2)
                def _():
                    per_parity(r - 2, lambda p: out_desc(
                        ch - 2 * _NW, p, sems_o[p]).wait())

                per_parity(r, lambda p: compute(r, p))
                per_parity(r, lambda p: out_desc(ch, p, sems_o[p]).start())
                return carry

            lax.fori_loop(0, n_r, r_body, 0)
            per_parity(n_r - 2, lambda p: out_desc(
                wid + (n_r - 2) * _NW, p, sems_o[p]).wait())
            per_parity(n_r - 1, lambda p: out_desc(
                wid + (n_r - 1) * _NW, p, sems_o[p]).wait())

        if tail:
            @pl.when(wid == 0)
            def _():
                trows = tail * D // 128
                pltpu.sync_copy(tail_hbm, tail_v.at[pl.ds(0, trows)])
                pltpu.sync_copy(
                    tail_v.at[pl.ds(0, trows)],
                    out_hbm.at[pl.ds(n_ch * CH * D // 128, trows)])

    return tr_kernel


@functools.cache
def _build(B, NNZ, V, D):
    L = 16
    bw = B // _NW
    rpq = 128 // D
    shf = (rpq - 1).bit_length()
    dshf = (D - 1).bit_length()
    mesh = plsc.VectorSubcoreMesh(core_axis_name="c", subcore_axis_name="s")

    @functools.partial(
        pl.kernel,
        out_type=jax.ShapeDtypeStruct((NNZ, D, B), jnp.float32),
        mesh=mesh,
        scratch_types=[
            pltpu.VMEM((NNZ, bw), jnp.int32),
            pltpu.VMEM((NNZ, bw), jnp.float32),
            pltpu.VMEM((NNZ * bw,), jnp.int32),
            pltpu.VMEM((NNZ, bw), jnp.int32),
            pltpu.VMEM((2, bw, 128), jnp.float32),
            pltpu.VMEM((2, D, bw), jnp.float32),
            pltpu.SemaphoreType.DMA,
            pltpu.SemaphoreType.DMA,
            pltpu.SemaphoreType.DMA,
            pltpu.SemaphoreType.DMA,
        ],
        compiler_params=pltpu.CompilerParams(needs_layout_passes=False),
    )
    def sc_kernel(xt_hbm, vt_hbm, table_hbm, out_hbm,
                  x_v, val_v, q_v, off_v, quad_v, out_v,
                  sem_g0, sem_g1, sem_o0, sem_o1):
        wid = lax.axis_index("s") * _NC + lax.axis_index("c")
        b0 = pl.multiple_of(wid * bw, 128)
        pltpu.sync_copy(xt_hbm.at[:, pl.ds(b0, bw)], x_v)
        pltpu.sync_copy(vt_hbm.at[:, pl.ds(b0, bw)], val_v)

        def fmt_body(n, carry):
            for k in range(bw // L):
                x16 = x_v[n, pl.ds(k * L, L)]
                q_v[pl.ds(n * bw + k * L, L)] = lax.shift_right_logical(x16, shf)
                off_v[n, pl.ds(k * L, L)] = lax.shift_left(jnp.bitwise_and(x16, rpq - 1), dshf)
            return carry

        lax.fori_loop(0, NNZ, fmt_body, 0)

        def gather_desc(n, p, sem):
            return pltpu.make_async_copy(
                table_hbm.at[q_v.at[pl.ds(n * bw, bw)]],
                quad_v.at[p],
                sem,
            )

        def out_desc(n, p, sem):
            return pltpu.make_async_copy(
                out_v.at[p],
                out_hbm.at[n, :, pl.ds(b0, bw)],
                sem,
            )

        def per_parity(c, fn):
            @pl.when(c % 2 == 0)
            def _():
                fn(0)

            @pl.when(c % 2 == 1)
            def _():
                fn(1)

        sems_g = (sem_g0, sem_g1)
        sems_o = (sem_o0, sem_o1)

        riot = lax.iota(jnp.int32, L)
        rowcs = [k * L + riot for k in range(bw // L)]

        def compute_p(n, p):
            for k in range(bw // L):
                off16 = off_v[n, pl.ds(k * L, L)]
                val16 = val_v[n, pl.ds(k * L, L)]
                for c in range(D):
                    seg = plsc.load_gather(
                        quad_v.at[p], [rowcs[k], off16 + c])
                    out_v[p, c, pl.ds(k * L, L)] = seg * val16

        def compute(n):
            per_parity(n, lambda p: compute_p(n, p))

        per_parity(0, lambda p: gather_desc(0, p, sems_g[p]).start())

        def n_body(n, carry):
            @pl.when(n < NNZ - 1)
            def _():
                per_parity(n + 1,
                           lambda p: gather_desc(n + 1, p, sems_g[p]).start())

            per_parity(n, lambda p: gather_desc(n, p, sems_g[p]).wait())

            @pl.when(n >= 2)
            def _():
                per_parity(n - 2,
                           lambda p: out_desc(n - 2, p, sems_o[p]).wait())

            compute(n)
            per_parity(n, lambda p: out_desc(n, p, sems_o[p]).start())
            return carry

        lax.fori_loop(0, NNZ, n_body, 0)
        per_parity(NNZ - 2, lambda p: out_desc(NNZ - 2, p, sems_o[p]).wait())
        per_parity(NNZ - 1, lambda p: out_desc(NNZ - 1, p, sems_o[p]).wait())

    return sc_kernel


def kernel(x, x_val, table):
    B, NNZ = x.shape
    V, D = table.shape
    rpq = 128 // D
    xt = jnp.transpose(x).astype(jnp.int32)
    vt = jnp.transpose(x_val)
    tt = jnp.transpose(table)
    n_ch = V // 512
    tail = V - n_ch * 512
    if tail:
        tail_in = table[n_ch * 512:].reshape(tail * D // 128, 128)
    else:
        tail_in = jnp.zeros((8, 128), jnp.float32)
    t4 = _build_transpose(V, D)(tt, tail_in)
    out_t = _build(B, NNZ, V, D)(xt, vt, t4)
    return jnp.transpose(out_t, (2, 0, 1))


# speedup vs baseline: 1.8274x; 1.2720x over previous
import functools

import jax
import jax.numpy as jnp
from jax import lax
from jax.experimental import pallas as pl
from jax.experimental.pallas import tpu as pltpu
from jax.experimental.pallas import tpu_sc as plsc

_NC = 2
_NS = 16
_NW = _NC * _NS


@functools.cache
def _build_transpose(V, D):
    L = 16
    CH = 512
    n_ch = V // CH
    tail = V - n_ch * CH
    mesh = plsc.VectorSubcoreMesh(core_axis_name="c", subcore_axis_name="s")

    @functools.partial(
        pl.kernel,
        out_type=jax.ShapeDtypeStruct((V * D // 128, 128), jnp.float32),
        mesh=mesh,
        scratch_types=[
            pltpu.VMEM((2, D, CH), jnp.float32),
            pltpu.VMEM((2, CH * D // 128, 128), jnp.float32),
            pltpu.VMEM((max(8, (V - (V // CH) * CH) * D // 128), 128),
                       jnp.float32),
            pltpu.SemaphoreType.DMA,
            pltpu.SemaphoreType.DMA,
            pltpu.SemaphoreType.DMA,
            pltpu.SemaphoreType.DMA,
        ],
        compiler_params=pltpu.CompilerParams(needs_layout_passes=False),
    )
    def tr_kernel(tt_hbm, tail_hbm, out_hbm, in_v, out_v, tail_v,
                  sem_i0, sem_i1, sem_o0, sem_o1):
        wid = lax.axis_index("s") * _NC + lax.axis_index("c")
        sems_i = (sem_i0, sem_i1)
        sems_o = (sem_o0, sem_o1)
        orows = CH * D // 128

        def in_desc(ch, p, sem):
            return pltpu.make_async_copy(
                tt_hbm.at[:, pl.ds(pl.multiple_of(ch * CH, 128), CH)],
                in_v.at[p], sem)

        def out_desc(ch, p, sem):
            return pltpu.make_async_copy(
                out_v.at[p],
                out_hbm.at[pl.ds(pl.multiple_of(ch * orows, 8), orows)], sem)

        def per_parity(c, fn):
            @pl.when(c % 2 == 0)
            def _():
                fn(0)

            @pl.when(c % 2 == 1)
            def _():
                fn(1)

        iot = lax.iota(jnp.int32, L)
        cvecs = [(m * L + iot) % D for m in range(128 // L)]
        avecs = [(m * L + iot) // D for m in range(128 // L)]
        UQ = 4

        def compute(r, p):
            @plsc.parallel_loop(0, orows, step=UQ, unroll=2)
            def qgrp(qm0):
                for u in range(UQ):
                    qm = qm0 + u
                    jbase = qm * (128 // D)
                    for m in range(128 // L):
                        seg = plsc.load_gather(
                            in_v.at[p], [cvecs[m], avecs[m] + jbase])
                        out_v[p, qm, pl.ds(m * L, L)] = seg

        n_r = (n_ch - wid + _NW - 1) // _NW

        @pl.when(n_r > 0)
        def _():
            per_parity(0, lambda p: in_desc(wid, p, sems_i[p]).start())

            def r_body(r, carry):
                ch = wid + r * _NW

                @pl.when(r + 1 < n_r)
                def _():
                    per_parity(r + 1, lambda p: in_desc(
                        ch + _NW, p, sems_i[p]).start())

                per_parity(r, lambda p: in_desc(ch, p, sems_i[p]).wait())

                @pl.when(r >= 2)
                def _():
                    per_parity(r - 2, lambda p: out_desc(
                        ch - 2 * _NW, p, sems_o[p]).wait())

                per_parity(r, lambda p: compute(r, p))
                per_parity(r, lambda p: out_desc(ch, p, sems_o[p]).start())
                return carry

            lax.fori_loop(0, n_r, r_body, 0)
            per_parity(n_r - 2, lambda p: out_desc(
                wid + (n_r - 2) * _NW, p, sems_o[p]).wait())
            per_parity(n_r - 1, lambda p: out_desc(
                wid + (n_r - 1) * _NW, p, sems_o[p]).wait())

        if tail:
            @pl.when(wid == 0)
            def _():
                trows = tail * D // 128
                pltpu.sync_copy(tail_hbm, tail_v.at[pl.ds(0, trows)])
                pltpu.sync_copy(
                    tail_v.at[pl.ds(0, trows)],
                    out_hbm.at[pl.ds(n_ch * CH * D // 128, trows)])

    return tr_kernel


@functools.cache
def _build(B, NNZ, V, D):
    L = 16
    bw = B // _NW
    rpq = 128 // D
    shf = (rpq - 1).bit_length()
    dshf = (D - 1).bit_length()
    mesh = plsc.VectorSubcoreMesh(core_axis_name="c", subcore_axis_name="s")

    @functools.partial(
        pl.kernel,
        out_type=jax.ShapeDtypeStruct((NNZ, D, B), jnp.float32),
        mesh=mesh,
        scratch_types=[
            pltpu.VMEM((NNZ, bw), jnp.int32),
            pltpu.VMEM((NNZ, bw), jnp.float32),
            pltpu.VMEM((NNZ * bw,), jnp.int32),
            pltpu.VMEM((NNZ, bw), jnp.int32),
            pltpu.VMEM((2, bw, 128), jnp.float32),
            pltpu.VMEM((2, D, bw), jnp.float32),
            pltpu.SemaphoreType.DMA,
            pltpu.SemaphoreType.DMA,
            pltpu.SemaphoreType.DMA,
            pltpu.SemaphoreType.DMA,
        ],
        compiler_params=pltpu.CompilerParams(needs_layout_passes=False),
    )
    def sc_kernel(xt_hbm, vt_hbm, table_hbm, out_hbm,
                  x_v, val_v, q_v, off_v, quad_v, out_v,
                  sem_g0, sem_g1, sem_o0, sem_o1):
        wid = lax.axis_index("s") * _NC + lax.axis_index("c")
        b0 = pl.multiple_of(wid * bw, 128)
        pltpu.sync_copy(xt_hbm.at[:, pl.ds(b0, bw)], x_v)
        pltpu.sync_copy(vt_hbm.at[:, pl.ds(b0, bw)], val_v)

        def fmt_body(n, carry):
            for k in range(bw // L):
                x16 = x_v[n, pl.ds(k * L, L)]
                q_v[pl.ds(n * bw + k * L, L)] = lax.shift_right_logical(x16, shf)
                off_v[n, pl.ds(k * L, L)] = lax.shift_left(jnp.bitwise_and(x16, rpq - 1), dshf)
            return carry

        lax.fori_loop(0, NNZ, fmt_body, 0)

        def gather_desc(n, p, sem):
            return pltpu.make_async_copy(
                table_hbm.at[q_v.at[pl.ds(n * bw, bw)]],
                quad_v.at[p],
                sem,
            )

        def out_desc(n, p, sem):
            return pltpu.make_async_copy(
                out_v.at[p],
                out_hbm.at[n, :, pl.ds(b0, bw)],
                sem,
            )

        def per_parity(c, fn):
            @pl.when(c % 2 == 0)
            def _():
                fn(0)

            @pl.when(c % 2 == 1)
            def _():
                fn(1)

        sems_g = (sem_g0, sem_g1)
        sems_o = (sem_o0, sem_o1)

        riot = lax.iota(jnp.int32, L)
        rowcs = [k * L + riot for k in range(bw // L)]

        def compute_p(n, p):
            for k in range(bw // L):
                off16 = off_v[n, pl.ds(k * L, L)]
                val16 = val_v[n, pl.ds(k * L, L)]
                rk = rowcs[k]

                @plsc.parallel_loop(0, D, step=1, unroll=8)
                def cgrp(c):
                    seg = plsc.load_gather(
                        quad_v.at[p], [rk, off16 + c])
                    out_v[p, c, pl.ds(k * L, L)] = seg * val16

        def compute(n):
            per_parity(n, lambda p: compute_p(n, p))

        per_parity(0, lambda p: gather_desc(0, p, sems_g[p]).start())

        def n_body(n, carry):
            @pl.when(n < NNZ - 1)
            def _():
                per_parity(n + 1,
                           lambda p: gather_desc(n + 1, p, sems_g[p]).start())

            per_parity(n, lambda p: gather_desc(n, p, sems_g[p]).wait())

            @pl.when(n >= 2)
            def _():
                per_parity(n - 2,
                           lambda p: out_desc(n - 2, p, sems_o[p]).wait())

            compute(n)
            per_parity(n, lambda p: out_desc(n, p, sems_o[p]).start())
            return carry

        lax.fori_loop(0, NNZ, n_body, 0)
        per_parity(NNZ - 2, lambda p: out_desc(NNZ - 2, p, sems_o[p]).wait())
        per_parity(NNZ - 1, lambda p: out_desc(NNZ - 1, p, sems_o[p]).wait())

    return sc_kernel


def kernel(x, x_val, table):
    B, NNZ = x.shape
    V, D = table.shape
    rpq = 128 // D
    xt = jnp.transpose(x).astype(jnp.int32)
    vt = jnp.transpose(x_val)
    tt = jnp.transpose(table)
    n_ch = V // 512
    tail = V - n_ch * 512
    if tail:
        tail_in = table[n_ch * 512:].reshape(tail * D // 128, 128)
    else:
        tail_in = jnp.zeros((8, 128), jnp.float32)
    t4 = _build_transpose(V, D)(tt, tail_in)
    out_t = _build(B, NNZ, V, D)(xt, vt, t4)
    return jnp.transpose(out_t, (2, 0, 1))


# speedup vs baseline: 1.8388x; 1.0062x over previous
import functools

import jax
import jax.numpy as jnp
from jax import lax
from jax.experimental import pallas as pl
from jax.experimental.pallas import tpu as pltpu
from jax.experimental.pallas import tpu_sc as plsc

_NC = 2
_NS = 16
_NW = _NC * _NS


@functools.cache
def _build_transpose(V, D):
    L = 16
    CH = 768
    n_ch = V // CH
    tail = V - n_ch * CH
    mesh = plsc.VectorSubcoreMesh(core_axis_name="c", subcore_axis_name="s")

    @functools.partial(
        pl.kernel,
        out_type=jax.ShapeDtypeStruct((V * D // 128, 128), jnp.float32),
        mesh=mesh,
        scratch_types=[
            pltpu.VMEM((2, D, CH), jnp.float32),
            pltpu.VMEM((2, CH * D // 128, 128), jnp.float32),
            pltpu.VMEM((max(8, (V - (V // CH) * CH) * D // 128), 128),
                       jnp.float32),
            pltpu.SemaphoreType.DMA,
            pltpu.SemaphoreType.DMA,
            pltpu.SemaphoreType.DMA,
            pltpu.SemaphoreType.DMA,
        ],
        compiler_params=pltpu.CompilerParams(needs_layout_passes=False),
    )
    def tr_kernel(tt_hbm, tail_hbm, out_hbm, in_v, out_v, tail_v,
                  sem_i0, sem_i1, sem_o0, sem_o1):
        wid = lax.axis_index("s") * _NC + lax.axis_index("c")
        sems_i = (sem_i0, sem_i1)
        sems_o = (sem_o0, sem_o1)
        orows = CH * D // 128

        def in_desc(ch, p, sem):
            return pltpu.make_async_copy(
                tt_hbm.at[:, pl.ds(pl.multiple_of(ch * CH, 128), CH)],
                in_v.at[p], sem)

        def out_desc(ch, p, sem):
            return pltpu.make_async_copy(
                out_v.at[p],
                out_hbm.at[pl.ds(pl.multiple_of(ch * orows, 8), orows)], sem)

        def per_parity(c, fn):
            @pl.when(c % 2 == 0)
            def _():
                fn(0)

            @pl.when(c % 2 == 1)
            def _():
                fn(1)

        iot = lax.iota(jnp.int32, L)
        cvecs = [(m * L + iot) % D for m in range(128 // L)]
        avecs = [(m * L + iot) // D for m in range(128 // L)]
        UQ = 4

        def compute(r, p):
            @plsc.parallel_loop(0, orows, step=UQ, unroll=2)
            def qgrp(qm0):
                for u in range(UQ):
                    qm = qm0 + u
                    jbase = qm * (128 // D)
                    for m in range(128 // L):
                        seg = plsc.load_gather(
                            in_v.at[p], [cvecs[m], avecs[m] + jbase])
                        out_v[p, qm, pl.ds(m * L, L)] = seg

        n_r = (n_ch - wid + _NW - 1) // _NW

        @pl.when(n_r > 0)
        def _():
            per_parity(0, lambda p: in_desc(wid, p, sems_i[p]).start())

            def r_body(r, carry):
                ch = wid + r * _NW

                @pl.when(r + 1 < n_r)
                def _():
                    per_parity(r + 1, lambda p: in_desc(
                        ch + _NW, p, sems_i[p]).start())

                per_parity(r, lambda p: in_desc(ch, p, sems_i[p]).wait())

                @pl.when(r >= 2)
                def _():
                    per_parity(r - 2, lambda p: out_desc(
                        ch - 2 * _NW, p, sems_o[p]).wait())

                per_parity(r, lambda p: compute(r, p))
                per_parity(r, lambda p: out_desc(ch, p, sems_o[p]).start())
                return carry

            lax.fori_loop(0, n_r, r_body, 0)
            per_parity(n_r - 2, lambda p: out_desc(
                wid + (n_r - 2) * _NW, p, sems_o[p]).wait())
            per_parity(n_r - 1, lambda p: out_desc(
                wid + (n_r - 1) * _NW, p, sems_o[p]).wait())

        if tail:
            @pl.when(wid == 0)
            def _():
                trows = tail * D // 128
                pltpu.sync_copy(tail_hbm, tail_v.at[pl.ds(0, trows)])
                pltpu.sync_copy(
                    tail_v.at[pl.ds(0, trows)],
                    out_hbm.at[pl.ds(n_ch * CH * D // 128, trows)])

    return tr_kernel


@functools.cache
def _build(B, NNZ, V, D):
    L = 16
    bw = B // _NW
    rpq = 128 // D
    shf = (rpq - 1).bit_length()
    dshf = (D - 1).bit_length()
    mesh = plsc.VectorSubcoreMesh(core_axis_name="c", subcore_axis_name="s")

    @functools.partial(
        pl.kernel,
        out_type=jax.ShapeDtypeStruct((NNZ, D, B), jnp.float32),
        mesh=mesh,
        scratch_types=[
            pltpu.VMEM((NNZ, bw), jnp.int32),
            pltpu.VMEM((NNZ, bw), jnp.float32),
            pltpu.VMEM((NNZ * bw,), jnp.int32),
            pltpu.VMEM((NNZ, bw), jnp.int32),
            pltpu.VMEM((2, bw, 128), jnp.float32),
            pltpu.VMEM((2, D, bw), jnp.float32),
            pltpu.SemaphoreType.DMA,
            pltpu.SemaphoreType.DMA,
            pltpu.SemaphoreType.DMA,
            pltpu.SemaphoreType.DMA,
        ],
        compiler_params=pltpu.CompilerParams(needs_layout_passes=False),
    )
    def sc_kernel(xt_hbm, vt_hbm, table_hbm, out_hbm,
                  x_v, val_v, q_v, off_v, quad_v, out_v,
                  sem_g0, sem_g1, sem_o0, sem_o1):
        wid = lax.axis_index("s") * _NC + lax.axis_index("c")
        b0 = pl.multiple_of(wid * bw, 128)
        pltpu.sync_copy(xt_hbm.at[:, pl.ds(b0, bw)], x_v)
        pltpu.sync_copy(vt_hbm.at[:, pl.ds(b0, bw)], val_v)

        def fmt_body(n, carry):
            for k in range(bw // L):
                x16 = x_v[n, pl.ds(k * L, L)]
                q_v[pl.ds(n * bw + k * L, L)] = lax.shift_right_logical(x16, shf)
                off_v[n, pl.ds(k * L, L)] = lax.shift_left(jnp.bitwise_and(x16, rpq - 1), dshf)
            return carry

        lax.fori_loop(0, NNZ, fmt_body, 0)

        def gather_desc(n, p, sem):
            return pltpu.make_async_copy(
                table_hbm.at[q_v.at[pl.ds(n * bw, bw)]],
                quad_v.at[p],
                sem,
            )

        def out_desc(n, p, sem):
            return pltpu.make_async_copy(
                out_v.at[p],
                out_hbm.at[n, :, pl.ds(b0, bw)],
                sem,
            )

        def per_parity(c, fn):
            @pl.when(c % 2 == 0)
            def _():
                fn(0)

            @pl.when(c % 2 == 1)
            def _():
                fn(1)

        sems_g = (sem_g0, sem_g1)
        sems_o = (sem_o0, sem_o1)

        riot = lax.iota(jnp.int32, L)
        rowcs = [k * L + riot for k in range(bw // L)]

        def compute_p(n, p):
            for k in range(bw // L):
                off16 = off_v[n, pl.ds(k * L, L)]
                val16 = val_v[n, pl.ds(k * L, L)]
                rk = rowcs[k]

                @plsc.parallel_loop(0, D, step=1, unroll=8)
                def cgrp(c):
                    seg = plsc.load_gather(
                        quad_v.at[p], [rk, off16 + c])
                    out_v[p, c, pl.ds(k * L, L)] = seg * val16

        def compute(n):
            per_parity(n, lambda p: compute_p(n, p))

        per_parity(0, lambda p: gather_desc(0, p, sems_g[p]).start())

        def n_body(n, carry):
            @pl.when(n < NNZ - 1)
            def _():
                per_parity(n + 1,
                           lambda p: gather_desc(n + 1, p, sems_g[p]).start())

            per_parity(n, lambda p: gather_desc(n, p, sems_g[p]).wait())

            @pl.when(n >= 2)
            def _():
                per_parity(n - 2,
                           lambda p: out_desc(n - 2, p, sems_o[p]).wait())

            compute(n)
            per_parity(n, lambda p: out_desc(n, p, sems_o[p]).start())
            return carry

        lax.fori_loop(0, NNZ, n_body, 0)
        per_parity(NNZ - 2, lambda p: out_desc(NNZ - 2, p, sems_o[p]).wait())
        per_parity(NNZ - 1, lambda p: out_desc(NNZ - 1, p, sems_o[p]).wait())

    return sc_kernel


def kernel(x, x_val, table):
    B, NNZ = x.shape
    V, D = table.shape
    rpq = 128 // D
    xt = jnp.transpose(x).astype(jnp.int32)
    vt = jnp.transpose(x_val)
    tt = jnp.transpose(table)
    n_ch = V // 768
    tail = V - n_ch * 768
    if tail:
        tail_in = table[n_ch * 768:].reshape(tail * D // 128, 128)
    else:
        tail_in = jnp.zeros((8, 128), jnp.float32)
    t4 = _build_transpose(V, D)(tt, tail_in)
    out_t = _build(B, NNZ, V, D)(xt, vt, t4)
    return jnp.transpose(out_t, (2, 0, 1))
